# baseline (speedup 1.0000x reference)
import functools

import jax
import jax.numpy as jnp
from jax import lax
from jax.experimental import pallas as pl
from jax.experimental.pallas import tpu as pltpu
from jax.experimental.pallas import tpu_sc as plsc


def _prep_kernel(emb_ref, w1_ref, t_ref):
    f32 = jnp.float32
    v, d = emb_ref.shape
    emb = emb_ref[...]
    nf = w1_ref.shape[1] // d
    t_ref[...] = jnp.zeros_like(t_ref)
    for i in range(nf):
        w1_i = w1_ref[:, i * d:(i + 1) * d]
        t_ref[i * v:(i + 1) * v, 0:d] = jax.lax.dot_general(
            emb, w1_i, (((1,), (1,)), ((), ())),
            preferred_element_type=f32)


def _mlp_kernel(h1p_ref, b1_ref, w2_ref, b2_ref, w3_ref, b3_ref, out_ref):
    f32 = jnp.float32
    x = h1p_ref[...] + b1_ref[...]
    alpha = 1.6732632423543772
    scale = 1.0507009873554805
    h1 = scale * jnp.where(x > 0, x, alpha * (jnp.exp(x) - 1.0))
    h2 = jnp.tanh(
        jax.lax.dot_general(h1, w2_ref[...], (((1,), (1,)), ((), ())),
                            preferred_element_type=f32) + b2_ref[...])
    out_ref[...] = (jnp.sum(h2 * w3_ref[...], axis=1, keepdims=True)
                    + b3_ref[0, 0])


def _make_sc_gather(b, ch):
    nw = 32
    b_per_w = b // nw
    nchunks = b_per_w // ch
    mesh = plsc.VectorSubcoreMesh(core_axis_name="c", subcore_axis_name="s")

    @functools.partial(
        pl.kernel, mesh=mesh,
        out_type=jax.ShapeDtypeStruct((b, 128), jnp.float32),
        scratch_types=[
            pltpu.VMEM((ch,), jnp.int32),
            pltpu.VMEM((ch, 128), jnp.float32),
            pltpu.VMEM((ch, 128), jnp.float32),
            pltpu.SemaphoreType.DMA,
        ],
    )
    def sc_gather(t_hbm, ids_hbm, out_hbm, idx_v, rows_v, acc_v, sem):
        nc = 2
        wid = lax.axis_index("s") * nc + lax.axis_index("c")
        base = wid * b_per_w

        def accumulate(i, carry):
            for k in range(8):
                sl = pl.ds(k * 16, 16)
                acc_v[i, sl] = acc_v[i, sl] + rows_v[i, sl]
            return carry

        for c in range(nchunks):
            off = base + c * ch
            pltpu.sync_copy(ids_hbm.at[pl.ds(off, ch)], idx_v)
            pltpu.async_copy(t_hbm.at[idx_v], acc_v, sem).wait()
            pltpu.sync_copy(ids_hbm.at[pl.ds(b + off, ch)], idx_v)
            pltpu.async_copy(t_hbm.at[idx_v], rows_v, sem).wait()
            lax.fori_loop(0, ch, accumulate, 0)
            pltpu.sync_copy(ids_hbm.at[pl.ds(2 * b + off, ch)], idx_v)
            pltpu.async_copy(t_hbm.at[idx_v], rows_v, sem).wait()
            lax.fori_loop(0, ch, accumulate, 0)
            pltpu.sync_copy(acc_v, out_hbm.at[pl.ds(off, ch)])

    return sc_gather


def kernel(vocab_ids, pos_emb, W1, b1, W2, b2, W3, b3):
    nf, b = vocab_ids.shape
    v, d = pos_emb.shape
    h = W1.shape[0]

    offs = (jnp.arange(nf, dtype=jnp.int32) * v)[:, None]
    ids_flat = (vocab_ids.astype(jnp.int32) + offs).reshape(-1)

    t_pad = pl.pallas_call(
        _prep_kernel,
        out_shape=jax.ShapeDtypeStruct((nf * v, 128), jnp.float32),
    )(pos_emb, W1)

    h1pre = _make_sc_gather(b, 256)(t_pad, ids_flat)

    b1p = jnp.pad(b1.reshape(1, -1), ((0, 0), (0, 128 - h)))
    w2p = jnp.pad(W2, ((0, 0), (0, 128 - d)))

    tb = 4096 if b % 4096 == 0 else b
    nb = b // tb
    return pl.pallas_call(
        _mlp_kernel,
        grid=(nb,),
        in_specs=[
            pl.BlockSpec((tb, 128), lambda i: (i, 0)),
            pl.BlockSpec((1, 128), lambda i: (0, 0)),
            pl.BlockSpec(w2p.shape, lambda i: (0, 0)),
            pl.BlockSpec((1, b2.shape[0]), lambda i: (0, 0)),
            pl.BlockSpec(W3.shape, lambda i: (0, 0)),
            pl.BlockSpec((1, 1), lambda i: (0, 0)),
        ],
        out_specs=pl.BlockSpec((tb, 1), lambda i: (i, 0)),
        out_shape=jax.ShapeDtypeStruct((b, 1), jnp.float32),
    )(h1pre, b1p, w2p, b2.reshape(1, -1), W3, b3.reshape(1, 1))


# speedup vs baseline: 5.8505x; 5.8505x over previous
import jax
import jax.numpy as jnp
from jax.experimental import pallas as pl
from jax.experimental.pallas import tpu as pltpu


def _fwd_kernel(ids_ref, emb_ref, w1_ref, b1bc_ref, w2a_ref, w3a_ref,
                out_ref, t_ref):
    f32 = jnp.float32
    bf16 = jnp.bfloat16
    nf, tb = ids_ref.shape
    v, d = emb_ref.shape
    h = w1_ref.shape[0]

    @pl.when(pl.program_id(0) == 0)
    def _precompute_tables():
        emb = emb_ref[...]
        for i in range(nf):
            w1_i = w1_ref[:, i * d:(i + 1) * d]
            tt = jax.lax.dot_general(w1_i, emb, (((1,), (1,)), ((), ())),
                                     preferred_element_type=f32)
            if i == 0:
                tt = tt + b1bc_ref[...]
            t_ref[:, i * v:(i + 1) * v] = tt.astype(bf16)

    ids = ids_ref[...]
    sub_iota = jax.lax.broadcasted_iota(jnp.int32, (v, tb), 0)
    oh = jnp.concatenate(
        [(ids[i:i + 1, :] == sub_iota).astype(bf16)
         for i in range(nf)], axis=0)
    x = jax.lax.dot_general(t_ref[...], oh, (((1,), (0,)), ((), ())),
                            preferred_element_type=f32)

    alpha = 1.6732632423543772
    scale = 1.0507009873554805
    h1 = scale * jnp.where(x > 0, x, alpha * (jnp.exp(x) - 1.0))
    ones_row = jnp.ones((1, tb), f32)
    h1a = jnp.concatenate([h1, ones_row], axis=0)
    h2 = jnp.tanh(
        jax.lax.dot_general(w2a_ref[...], h1a, (((1,), (0,)), ((), ())),
                            preferred_element_type=f32))
    h2a = jnp.concatenate([h2, ones_row], axis=0)
    o8 = jax.lax.dot_general(w3a_ref[...], h2a, (((1,), (0,)), ((), ())),
                             preferred_element_type=f32)
    out_ref[...] = o8[0:1, :]


def kernel(vocab_ids, pos_emb, W1, b1, W2, b2, W3, b3):
    nf, b = vocab_ids.shape
    v, d = pos_emb.shape
    h = W1.shape[0]
    ids = vocab_ids.astype(jnp.int32)
    b1bc = jnp.broadcast_to(b1[:, None], (h, v))
    w2a = jnp.concatenate([W2, b2[:, None]], axis=1)
    w3a = jnp.pad(jnp.concatenate([W3, b3[:, None]], axis=1),
                  ((0, 7), (0, 0)))
    tb = 4096 if b % 4096 == 0 else b
    nb = b // tb
    out_row = pl.pallas_call(
        _fwd_kernel,
        grid=(nb,),
        in_specs=[
            pl.BlockSpec((nf, tb), lambda i: (0, i)),
            pl.BlockSpec(pos_emb.shape, lambda i: (0, 0)),
            pl.BlockSpec(W1.shape, lambda i: (0, 0)),
            pl.BlockSpec((h, v), lambda i: (0, 0)),
            pl.BlockSpec(w2a.shape, lambda i: (0, 0)),
            pl.BlockSpec(w3a.shape, lambda i: (0, 0)),
        ],
        out_specs=pl.BlockSpec((1, tb), lambda i: (0, i)),
        out_shape=jax.ShapeDtypeStruct((1, b), jnp.float32),
        scratch_shapes=[pltpu.VMEM((h, nf * v), jnp.bfloat16)],
    )(ids, pos_emb, W1, b1bc, w2a, w3a)
    return out_row.reshape(b, 1)
